# baseline (speedup 1.0000x reference)
import functools

import jax
import jax.numpy as jnp
from jax import lax
from jax.experimental import pallas as pl
from jax.experimental.pallas import tpu as pltpu
from jax.experimental.pallas import tpu_sc as plsc

D_MODEL = 1024
D_FF = 2048
NUM_EXPERTS = 8
SEQ = 2048
NPAIR = SEQ * 2

TILE = 256
NT = NPAIR // TILE + NUM_EXPERTS - 1
PROWS = NT * TILE

SC_CORES = 2
SC_SUBCORES = 16
NWORKERS = SC_CORES * SC_SUBCORES
CHUNK = SEQ // NWORKERS
SUBCH = CHUNK // 2
LANES = 16


def _gate_route_body(x_ref, gw_ref, s0_ref, s1_ref, te_ref, nt_ref, lb_ref):
    xf = x_ref[...]
    gw = gw_ref[...]
    logits = lax.dot_general(
        xf, gw, (((1,), (1,)), ((), ())),
        preferred_element_type=jnp.float32,
        precision=lax.Precision.DEFAULT)

    m = jnp.max(logits, axis=1, keepdims=True)
    p = jnp.exp(logits - m)
    scores = p / jnp.sum(p, axis=1, keepdims=True)
    ep = jnp.sum(scores, axis=0, keepdims=True) * (1.0 / SEQ)
    lb_ref[0, 0] = jnp.sum(ep * jnp.log(ep + 1e-8))

    i8 = lax.broadcasted_iota(jnp.int32, (SEQ, NUM_EXPERTS), 1)
    m0 = jnp.max(scores, axis=1, keepdims=True)
    e0 = jnp.min(jnp.where(scores == m0, i8, NUM_EXPERTS), axis=1,
                 keepdims=True)
    l2 = jnp.where(i8 == e0, -1.0, scores)
    m1 = jnp.max(l2, axis=1, keepdims=True)
    e1 = jnp.min(jnp.where(l2 == m1, i8, NUM_EXPERTS), axis=1,
                 keepdims=True)
    oh0 = (i8 == e0).astype(jnp.float32)
    oh1 = (i8 == e1).astype(jnp.float32)
    oh = oh0 + oh1

    cnt = jnp.sum(oh, axis=0, keepdims=True)
    ntl = jnp.floor((cnt + (TILE - 1)) * (1.0 / TILE))
    iu = lax.broadcasted_iota(jnp.int32, (NUM_EXPERTS, NUM_EXPERTS), 0)
    ju = lax.broadcasted_iota(jnp.int32, (NUM_EXPERTS, NUM_EXPERTS), 1)
    upper = (iu < ju).astype(jnp.float32)
    tileoff = lax.dot_general(ntl, upper, (((1,), (0,)), ((), ())),
                              preferred_element_type=jnp.float32)
    rowoff = tileoff * TILE
    total = jnp.sum(ntl)
    nt_ref[0, 0] = total.astype(jnp.int32)

    ends = tileoff + ntl
    jt = lax.broadcasted_iota(jnp.int32, (NT, NUM_EXPERTS), 0).astype(
        jnp.float32)
    te_raw = jnp.sum((ends <= jt).astype(jnp.float32), axis=1,
                     keepdims=True)
    i1x8 = lax.broadcasted_iota(jnp.int32, (1, NUM_EXPERTS), 1).astype(
        jnp.float32)
    last_e = jnp.max(i1x8 * (cnt > 0).astype(jnp.float32))
    te_ref[...] = jnp.minimum(te_raw, last_e).astype(jnp.int32)

    B = 128
    ri = lax.broadcasted_iota(jnp.int32, (B, B), 0)
    ci = lax.broadcasted_iota(jnp.int32, (B, B), 1)
    lstrict = (ri > ci).astype(jnp.float32)
    prefix = jnp.zeros((1, NUM_EXPERTS), jnp.float32)
    for g in range(SEQ // B):
        ohg = oh[g * B:(g + 1) * B, :]
        oh0g = oh0[g * B:(g + 1) * B, :]
        oh1g = oh1[g * B:(g + 1) * B, :]
        rg = lax.dot_general(lstrict, ohg, (((1,), (0,)), ((), ())),
                             preferred_element_type=jnp.float32)
        slotg = rg + prefix + rowoff
        s0g = jnp.sum(oh0g * slotg, axis=1, keepdims=True)
        s1g = jnp.sum(oh1g * slotg, axis=1, keepdims=True)
        s0_ref[g * B:(g + 1) * B, :] = s0g.astype(jnp.int32)
        s1_ref[g * B:(g + 1) * B, :] = s1g.astype(jnp.int32)
        prefix = prefix + jnp.sum(ohg, axis=0, keepdims=True)


def _gate_route(xf, gate_w, *, interpret=False):
    return pl.pallas_call(
        _gate_route_body,
        out_shape=(
            jax.ShapeDtypeStruct((SEQ, 1), jnp.int32),
            jax.ShapeDtypeStruct((SEQ, 1), jnp.int32),
            jax.ShapeDtypeStruct((NT, 1), jnp.int32),
            jax.ShapeDtypeStruct((1, 1), jnp.int32),
            jax.ShapeDtypeStruct((1, 1), jnp.float32),
        ),
        out_specs=(
            pl.BlockSpec(memory_space=pltpu.VMEM),
            pl.BlockSpec(memory_space=pltpu.VMEM),
            pl.BlockSpec(memory_space=pltpu.VMEM),
            pl.BlockSpec(memory_space=pltpu.SMEM),
            pl.BlockSpec(memory_space=pltpu.SMEM),
        ),
        interpret=interpret,
    )(xf, gate_w)


def _dispatch_body(x_hbm, s0_hbm, s1_hbm, xs_hbm, xv, i0v, i1v, sem0, sem1):
    w = lax.axis_index("s") * SC_CORES + lax.axis_index("c")
    base = w * CHUNK
    pltpu.sync_copy(s0_hbm.at[pl.ds(base, CHUNK)], i0v)
    pltpu.sync_copy(s1_hbm.at[pl.ds(base, CHUNK)], i1v)
    pltpu.sync_copy(x_hbm.at[pl.ds(base, CHUNK)], xv)
    c0 = pltpu.async_copy(xv, xs_hbm.at[i0v], sem0)
    c1 = pltpu.async_copy(xv, xs_hbm.at[i1v], sem1)
    c0.wait()
    c1.wait()


def _dispatch(xf, s0, s1):
    mesh = plsc.VectorSubcoreMesh(core_axis_name="c", subcore_axis_name="s")
    return pl.kernel(
        _dispatch_body,
        out_type=jax.ShapeDtypeStruct((PROWS, D_MODEL), jnp.float32),
        mesh=mesh,
        scratch_types=[
            pltpu.VMEM((CHUNK, D_MODEL), jnp.float32),
            pltpu.VMEM((CHUNK,), jnp.int32),
            pltpu.VMEM((CHUNK,), jnp.int32),
            pltpu.SemaphoreType.DMA,
            pltpu.SemaphoreType.DMA,
        ],
    )(xf, s0, s1)


NF = 2
FCH = D_FF // NF


def _ffn_body(te_ref, nt_ref, xs_ref, w1_ref, w3_ref, w2_ref, ys_ref):
    j = pl.program_id(0)
    f = pl.program_id(1)

    @pl.when(j < nt_ref[0])
    def _():
        a = xs_ref[...]
        w1b = w1_ref[0]
        w3b = w3_ref[0]
        w2b = w2_ref[0]
        h = lax.dot_general(a, w1b, (((1,), (1,)), ((), ())),
                            preferred_element_type=jnp.float32)
        v = lax.dot_general(a, w3b, (((1,), (1,)), ((), ())),
                            preferred_element_type=jnp.float32)
        g = (h / (1.0 + jnp.exp(-h))) * v
        part = lax.dot_general(g, w2b, (((1,), (1,)), ((), ())),
                               preferred_element_type=jnp.float32)

        @pl.when(f == 0)
        def _():
            ys_ref[...] = part

        @pl.when(f != 0)
        def _():
            ys_ref[...] += part


def _ffn(te, ntile, xs, w1, w3, w2, *, interpret=False):
    grid_spec = pltpu.PrefetchScalarGridSpec(
        num_scalar_prefetch=2,
        grid=(NT, NF),
        in_specs=[
            pl.BlockSpec((TILE, D_MODEL), lambda j, f, te, nt: (j, 0)),
            pl.BlockSpec((1, FCH, D_MODEL),
                         lambda j, f, te, nt: (te[j], f, 0)),
            pl.BlockSpec((1, FCH, D_MODEL),
                         lambda j, f, te, nt: (te[j], f, 0)),
            pl.BlockSpec((1, D_MODEL, FCH),
                         lambda j, f, te, nt: (te[j], 0, f)),
        ],
        out_specs=pl.BlockSpec((TILE, D_MODEL), lambda j, f, te, nt: (j, 0)),
    )
    return pl.pallas_call(
        _ffn_body,
        grid_spec=grid_spec,
        out_shape=jax.ShapeDtypeStruct((PROWS, D_MODEL), jnp.float32),
        interpret=interpret,
    )(te, ntile, xs, w1, w3, w2)


def _combine_body(ys_hbm, s0_hbm, s1_hbm, out_hbm, b0, b1, i0v, i1v,
                  sem0, sem1):
    w = lax.axis_index("s") * SC_CORES + lax.axis_index("c")
    for sub in range(CHUNK // SUBCH):
        base = w * CHUNK + sub * SUBCH
        pltpu.sync_copy(s0_hbm.at[pl.ds(base, SUBCH)], i0v)
        pltpu.sync_copy(s1_hbm.at[pl.ds(base, SUBCH)], i1v)
        c0 = pltpu.async_copy(ys_hbm.at[i0v], b0, sem0)
        c1 = pltpu.async_copy(ys_hbm.at[i1v], b1, sem1)
        c0.wait()
        c1.wait()

        def add_row(r, _):
            for cc in range(D_MODEL // LANES):
                sl = pl.ds(cc * LANES, LANES)
                b0[r, sl] = b0[r, sl] + b1[r, sl]
            return 0

        lax.fori_loop(0, SUBCH, add_row, 0)
        pltpu.sync_copy(b0, out_hbm.at[pl.ds(base, SUBCH)])


def _combine(ys, s0, s1):
    mesh = plsc.VectorSubcoreMesh(core_axis_name="c", subcore_axis_name="s")
    return pl.kernel(
        _combine_body,
        out_type=jax.ShapeDtypeStruct((SEQ, D_MODEL), jnp.float32),
        mesh=mesh,
        scratch_types=[
            pltpu.VMEM((SUBCH, D_MODEL), jnp.float32),
            pltpu.VMEM((SUBCH, D_MODEL), jnp.float32),
            pltpu.VMEM((SUBCH,), jnp.int32),
            pltpu.VMEM((SUBCH,), jnp.int32),
            pltpu.SemaphoreType.DMA,
            pltpu.SemaphoreType.DMA,
        ],
    )(ys, s0, s1)


def kernel(x, gate_w, w1, w2, w3):
    b, s, d = x.shape
    xf = x.reshape(s, d)
    s0, s1, te, ntile, lb = _gate_route(xf, gate_w)
    s0 = s0.reshape(-1)
    s1 = s1.reshape(-1)
    xs = _dispatch(xf, s0, s1)
    ys = _ffn(te.reshape(-1), ntile.reshape(-1), xs, w1, w3, w2)
    out = _combine(ys, s0, s1)
    return out.reshape(b, s, d), lb.reshape(())


# speedup vs baseline: 1.3976x; 1.3976x over previous
import functools

import jax
import jax.numpy as jnp
from jax import lax
from jax.experimental import pallas as pl
from jax.experimental.pallas import tpu as pltpu
from jax.experimental.pallas import tpu_sc as plsc

D_MODEL = 1024
D_FF = 2048
NUM_EXPERTS = 8
SEQ = 2048
NPAIR = SEQ * 2

TILE = 256
NT = NPAIR // TILE + NUM_EXPERTS - 1
PROWS = NT * TILE

SC_CORES = 2
SC_SUBCORES = 16
NWORKERS = SC_CORES * SC_SUBCORES
CHUNK = SEQ // NWORKERS
SUBCH = CHUNK // 2
LANES = 16


def _gate_route_body(x_ref, gw_ref, s0_ref, s1_ref, te_ref, nt_ref, lb_ref):
    xf = x_ref[...]
    gw = gw_ref[...]
    logits = lax.dot_general(
        xf, gw, (((1,), (1,)), ((), ())),
        preferred_element_type=jnp.float32,
        precision=lax.Precision.DEFAULT)

    m = jnp.max(logits, axis=1, keepdims=True)
    p = jnp.exp(logits - m)
    scores = p / jnp.sum(p, axis=1, keepdims=True)
    ep = jnp.sum(scores, axis=0, keepdims=True) * (1.0 / SEQ)
    lb_ref[0, 0] = jnp.sum(ep * jnp.log(ep + 1e-8))

    i8 = lax.broadcasted_iota(jnp.int32, (SEQ, NUM_EXPERTS), 1)
    m0 = jnp.max(scores, axis=1, keepdims=True)
    e0 = jnp.min(jnp.where(scores == m0, i8, NUM_EXPERTS), axis=1,
                 keepdims=True)
    l2 = jnp.where(i8 == e0, -1.0, scores)
    m1 = jnp.max(l2, axis=1, keepdims=True)
    e1 = jnp.min(jnp.where(l2 == m1, i8, NUM_EXPERTS), axis=1,
                 keepdims=True)
    oh0 = (i8 == e0).astype(jnp.float32)
    oh1 = (i8 == e1).astype(jnp.float32)
    oh = oh0 + oh1

    cnt = jnp.sum(oh, axis=0, keepdims=True)
    ntl = jnp.floor((cnt + (TILE - 1)) * (1.0 / TILE))
    iu = lax.broadcasted_iota(jnp.int32, (NUM_EXPERTS, NUM_EXPERTS), 0)
    ju = lax.broadcasted_iota(jnp.int32, (NUM_EXPERTS, NUM_EXPERTS), 1)
    upper = (iu < ju).astype(jnp.float32)
    tileoff = lax.dot_general(ntl, upper, (((1,), (0,)), ((), ())),
                              preferred_element_type=jnp.float32)
    rowoff = tileoff * TILE
    total = jnp.sum(ntl)
    nt_ref[0, 0] = total.astype(jnp.int32)

    ends = tileoff + ntl
    jt = lax.broadcasted_iota(jnp.int32, (NT, NUM_EXPERTS), 0).astype(
        jnp.float32)
    te_raw = jnp.sum((ends <= jt).astype(jnp.float32), axis=1,
                     keepdims=True)
    i1x8 = lax.broadcasted_iota(jnp.int32, (1, NUM_EXPERTS), 1).astype(
        jnp.float32)
    last_e = jnp.max(i1x8 * (cnt > 0).astype(jnp.float32))
    te_ref[...] = jnp.minimum(te_raw, last_e).astype(jnp.int32)

    B = 128
    ri = lax.broadcasted_iota(jnp.int32, (B, B), 0)
    ci = lax.broadcasted_iota(jnp.int32, (B, B), 1)
    lstrict = (ri > ci).astype(jnp.float32)
    prefix = jnp.zeros((1, NUM_EXPERTS), jnp.float32)
    for g in range(SEQ // B):
        ohg = oh[g * B:(g + 1) * B, :]
        oh0g = oh0[g * B:(g + 1) * B, :]
        oh1g = oh1[g * B:(g + 1) * B, :]
        rg = lax.dot_general(lstrict, ohg, (((1,), (0,)), ((), ())),
                             preferred_element_type=jnp.float32)
        slotg = rg + prefix + rowoff
        s0g = jnp.sum(oh0g * slotg, axis=1, keepdims=True)
        s1g = jnp.sum(oh1g * slotg, axis=1, keepdims=True)
        s0_ref[g * B:(g + 1) * B, :] = s0g.astype(jnp.int32)
        s1_ref[g * B:(g + 1) * B, :] = s1g.astype(jnp.int32)
        prefix = prefix + jnp.sum(ohg, axis=0, keepdims=True)


def _gate_route(xf, gate_w, *, interpret=False):
    return pl.pallas_call(
        _gate_route_body,
        out_shape=(
            jax.ShapeDtypeStruct((SEQ, 1), jnp.int32),
            jax.ShapeDtypeStruct((SEQ, 1), jnp.int32),
            jax.ShapeDtypeStruct((NT, 1), jnp.int32),
            jax.ShapeDtypeStruct((1, 1), jnp.int32),
            jax.ShapeDtypeStruct((1, 1), jnp.float32),
        ),
        out_specs=(
            pl.BlockSpec(memory_space=pltpu.VMEM),
            pl.BlockSpec(memory_space=pltpu.VMEM),
            pl.BlockSpec(memory_space=pltpu.VMEM),
            pl.BlockSpec(memory_space=pltpu.SMEM),
            pl.BlockSpec(memory_space=pltpu.SMEM),
        ),
        interpret=interpret,
    )(xf, gate_w)


def _dispatch_body(x_hbm, s0_hbm, s1_hbm, xs_hbm, xv, i0v, i1v, sem0, sem1,
                   semx):
    w = lax.axis_index("s") * SC_CORES + lax.axis_index("c")
    base = w * CHUNK
    la = pltpu.async_copy(s0_hbm.at[pl.ds(base, CHUNK)], i0v, sem0)
    lb = pltpu.async_copy(s1_hbm.at[pl.ds(base, CHUNK)], i1v, sem1)
    lx = pltpu.async_copy(x_hbm.at[pl.ds(base, CHUNK)], xv, semx)
    la.wait()
    lb.wait()
    lx.wait()
    c0 = pltpu.async_copy(xv, xs_hbm.at[i0v], sem0)
    c1 = pltpu.async_copy(xv, xs_hbm.at[i1v], sem1)
    c0.wait()
    c1.wait()


def _dispatch(xf, s0, s1):
    mesh = plsc.VectorSubcoreMesh(core_axis_name="c", subcore_axis_name="s")
    return pl.kernel(
        _dispatch_body,
        out_type=jax.ShapeDtypeStruct((PROWS, D_MODEL), jnp.float32),
        mesh=mesh,
        scratch_types=[
            pltpu.VMEM((CHUNK, D_MODEL), jnp.float32),
            pltpu.VMEM((CHUNK,), jnp.int32),
            pltpu.VMEM((CHUNK,), jnp.int32),
            pltpu.SemaphoreType.DMA,
            pltpu.SemaphoreType.DMA,
            pltpu.SemaphoreType.DMA,
        ],
    )(xf, s0, s1)


def _ffn_body(te_ref, nt_ref, xs_ref, w1_ref, w3_ref, w2_ref, ys_ref):
    j = pl.program_id(0)

    @pl.when(j < nt_ref[0])
    def _():
        a = xs_ref[...]
        w1b = w1_ref[0]
        w3b = w3_ref[0]
        w2b = w2_ref[0]
        h = lax.dot_general(a, w1b, (((1,), (1,)), ((), ())),
                            preferred_element_type=jnp.float32)
        v = lax.dot_general(a, w3b, (((1,), (1,)), ((), ())),
                            preferred_element_type=jnp.float32)
        g = (h / (1.0 + jnp.exp(-h))) * v
        ys_ref[...] = lax.dot_general(g, w2b, (((1,), (1,)), ((), ())),
                                      preferred_element_type=jnp.float32)


def _ffn(te, ntile, xs, w1, w3, w2, *, interpret=False):
    grid_spec = pltpu.PrefetchScalarGridSpec(
        num_scalar_prefetch=2,
        grid=(NT,),
        in_specs=[
            pl.BlockSpec((TILE, D_MODEL), lambda j, te, nt: (j, 0)),
            pl.BlockSpec((1, D_FF, D_MODEL), lambda j, te, nt: (te[j], 0, 0)),
            pl.BlockSpec((1, D_FF, D_MODEL), lambda j, te, nt: (te[j], 0, 0)),
            pl.BlockSpec((1, D_MODEL, D_FF), lambda j, te, nt: (te[j], 0, 0)),
        ],
        out_specs=pl.BlockSpec((TILE, D_MODEL), lambda j, te, nt: (j, 0)),
    )
    return pl.pallas_call(
        _ffn_body,
        grid_spec=grid_spec,
        out_shape=jax.ShapeDtypeStruct((PROWS, D_MODEL), jnp.float32),
        interpret=interpret,
    )(te, ntile, xs, w1, w3, w2)


NSUB = 4
SUB = CHUNK // NSUB


def _combine_body(ys_hbm, s0_hbm, s1_hbm, out_hbm, b0a, b1a, b0b, b1b,
                  i0v, i1v, g0a, g1a, g0b, g1b):
    w = lax.axis_index("s") * SC_CORES + lax.axis_index("c")
    base = w * CHUNK
    la = pltpu.async_copy(s0_hbm.at[pl.ds(base, CHUNK)], i0v, g0a)
    lb = pltpu.async_copy(s1_hbm.at[pl.ds(base, CHUNK)], i1v, g1a)
    la.wait()
    lb.wait()

    def start(s):
        b0, b1, ga, gb = ((b0a, b1a, g0a, g1a) if s % 2 == 0
                          else (b0b, b1b, g0b, g1b))
        c0 = pltpu.async_copy(ys_hbm.at[i0v.at[pl.ds(s * SUB, SUB)]], b0, ga)
        c1 = pltpu.async_copy(ys_hbm.at[i1v.at[pl.ds(s * SUB, SUB)]], b1, gb)
        return c0, c1

    pend = start(0)
    for s in range(NSUB):
        c0, c1 = pend
        c0.wait()
        c1.wait()
        if s + 1 < NSUB:
            pend = start(s + 1)
        b0, b1 = (b0a, b1a) if s % 2 == 0 else (b0b, b1b)

        def add_col(cc, _):
            sl = pl.ds(cc * LANES, LANES)
            for r in range(SUB):
                b0[r, sl] = b0[r, sl] + b1[r, sl]
            return 0

        lax.fori_loop(0, D_MODEL // LANES, add_col, 0)
        pltpu.sync_copy(b0, out_hbm.at[pl.ds(base + s * SUB, SUB)])


def _combine(ys, s0, s1):
    mesh = plsc.VectorSubcoreMesh(core_axis_name="c", subcore_axis_name="s")
    return pl.kernel(
        _combine_body,
        out_type=jax.ShapeDtypeStruct((SEQ, D_MODEL), jnp.float32),
        mesh=mesh,
        scratch_types=[
            pltpu.VMEM((SUB, D_MODEL), jnp.float32),
            pltpu.VMEM((SUB, D_MODEL), jnp.float32),
            pltpu.VMEM((SUB, D_MODEL), jnp.float32),
            pltpu.VMEM((SUB, D_MODEL), jnp.float32),
            pltpu.VMEM((CHUNK,), jnp.int32),
            pltpu.VMEM((CHUNK,), jnp.int32),
            pltpu.SemaphoreType.DMA,
            pltpu.SemaphoreType.DMA,
            pltpu.SemaphoreType.DMA,
            pltpu.SemaphoreType.DMA,
        ],
    )(ys, s0, s1)


def kernel(x, gate_w, w1, w2, w3):
    b, s, d = x.shape
    xf = x.reshape(s, d)
    s0, s1, te, ntile, lb = _gate_route(xf, gate_w)
    s0 = s0.reshape(-1)
    s1 = s1.reshape(-1)
    xs = _dispatch(xf, s0, s1)
    ys = _ffn(te.reshape(-1), ntile.reshape(-1), xs, w1, w3, w2)
    out = _combine(ys, s0, s1)
    return out.reshape(b, s, d), lb.reshape(())
